# Initial kernel scaffold; baseline (speedup 1.0000x reference)
import jax
import jax.numpy as jnp
from jax.experimental import pallas as pl


def kernel(input_x, edge_index, W0, b0, W1, b1, conv_w):
    raise NotImplementedError("write your pallas kernel here")


# speedup vs baseline: 5.1602x; 5.1602x over previous
import functools
import math

import jax
import jax.numpy as jnp
from jax import lax
from jax.experimental import pallas as pl
from jax.experimental.pallas import tpu as pltpu
from jax.experimental.pallas import tpu_sc as plsc

_N = 50000
_E = 800000
_D = 64
_H = 32
_ALPHA = 0.1
_THETA = 0.5
_LAYERS = 4

_NC = 2
_NS = 16
_CHUNK = 128
_INNER = 8
_BPT = 49
_CPT = _BPT * _INNER
_EPT = _CPT * _CHUNK
_EPAD = _EPT * _NS
_EB = _EPAD // _CHUNK
_NPT = 3136
_NPAD = _NPT * _NS
_ZR = 196
_BN = 2000


def _segment_sum_sc(x2, src2, dst2):
    mesh = plsc.VectorSubcoreMesh(core_axis_name="c", subcore_axis_name="s")

    @functools.partial(
        pl.kernel,
        out_type=jax.ShapeDtypeStruct((_NC, _N, _H), jnp.float32),
        mesh=mesh,
        scratch_types=[
            pltpu.VMEM_SHARED((_NPAD, _H), jnp.float32),
            pltpu.VMEM((_ZR, _H), jnp.float32),
            pltpu.VMEM((_INNER, _CHUNK), jnp.int32),
            pltpu.VMEM((_INNER, _CHUNK), jnp.int32),
            pltpu.VMEM((_CHUNK, _H), jnp.float32),
            pltpu.SemaphoreType.DMA,
        ],
        compiler_params=pltpu.CompilerParams(use_tc_tiling_on_sc=False),
    )
    def run(x2_hbm, src_hbm, dst_hbm, out_hbm, acc, zbuf, src_v, dst_v, rows_v, sem):
        c = lax.axis_index("c")
        s = lax.axis_index("s")
        zero16 = jnp.zeros((16,), jnp.float32)

        @pl.loop(0, _ZR)
        def _(i):
            zbuf[i, pl.ds(0, 16)] = zero16
            zbuf[i, pl.ds(16, 16)] = zero16

        @pl.loop(0, _NPT // _ZR)
        def _(j):
            pltpu.sync_copy(zbuf, acc.at[pl.ds(s * _NPT + j * _ZR, _ZR), :])

        plsc.subcore_barrier()

        @pl.loop(0, _BPT)
        def _(b):
            row0 = s * _CPT + b * _INNER
            pltpu.sync_copy(src_hbm.at[c, pl.ds(row0, _INNER), :], src_v)
            pltpu.sync_copy(dst_hbm.at[pl.ds(row0, _INNER), :], dst_v)
            for j in range(_INNER):
                pltpu.async_copy(x2_hbm.at[src_v.at[j]], rows_v, sem).wait()
                pltpu.sync_copy(rows_v, acc.at[dst_v.at[j]], add=True)

        plsc.subcore_barrier()

        last = _N - (_NS - 1) * _NPT

        @pl.when(s < _NS - 1)
        def _():
            pltpu.sync_copy(acc.at[pl.ds(s * _NPT, _NPT), :],
                            out_hbm.at[c, pl.ds(s * _NPT, _NPT), :])

        @pl.when(s == _NS - 1)
        def _():
            pltpu.sync_copy(acc.at[pl.ds((_NS - 1) * _NPT, last), :],
                            out_hbm.at[c, pl.ds((_NS - 1) * _NPT, last), :])

    return run(x2, src2, dst2)


def _proj_tc(input_x, W0, b0):
    def body(x_ref, w_ref, b_ref, out_ref):
        y = jnp.dot(x_ref[...], w_ref[...], preferred_element_type=jnp.float32)
        y = jnp.maximum(y + b_ref[...], 0.0)
        out_ref[0] = y[:, :_H]
        out_ref[1] = y[:, _H:]

    return pl.pallas_call(
        body,
        grid=(_N // _BN,),
        in_specs=[
            pl.BlockSpec((_BN, _D), lambda i: (i, 0)),
            pl.BlockSpec((_D, _D), lambda i: (0, 0)),
            pl.BlockSpec((1, _D), lambda i: (0, 0)),
        ],
        out_specs=pl.BlockSpec((_NC, _BN, _H), lambda i: (0, i, 0)),
        out_shape=jax.ShapeDtypeStruct((_NC, _N, _H), jnp.float32),
    )(input_x, W0, b0.reshape(1, _D))


def _combine_tc(agg2, x02, w, beta):
    def body(a_ref, x0_ref, w_ref, out_ref):
        agg = jnp.concatenate([a_ref[0], a_ref[1]], axis=1)
        x0 = jnp.concatenate([x0_ref[0], x0_ref[1]], axis=1)
        h = (1.0 - _ALPHA) * agg + _ALPHA * x0
        hw = jnp.dot(h, w_ref[...], preferred_element_type=jnp.float32)
        y = jnp.maximum((1.0 - beta) * h + beta * hw, 0.0)
        out_ref[0] = y[:, :_H]
        out_ref[1] = y[:, _H:]

    return pl.pallas_call(
        body,
        grid=(_N // _BN,),
        in_specs=[
            pl.BlockSpec((_NC, _BN, _H), lambda i: (0, i, 0)),
            pl.BlockSpec((_NC, _BN, _H), lambda i: (0, i, 0)),
            pl.BlockSpec((_D, _D), lambda i: (0, 0)),
        ],
        out_specs=pl.BlockSpec((_NC, _BN, _H), lambda i: (0, i, 0)),
        out_shape=jax.ShapeDtypeStruct((_NC, _N, _H), jnp.float32),
    )(agg2, x02, w)


def _final_tc(agg2, x02, w, W1, b1, beta):
    grid = _N // _BN

    def body(a_ref, x0_ref, w_ref, w1_ref, b1_ref, out_ref, acc_ref):
        i = pl.program_id(0)
        agg = jnp.concatenate([a_ref[0], a_ref[1]], axis=1)
        x0 = jnp.concatenate([x0_ref[0], x0_ref[1]], axis=1)
        h = (1.0 - _ALPHA) * agg + _ALPHA * x0
        hw = jnp.dot(h, w_ref[...], preferred_element_type=jnp.float32)
        y = jnp.maximum((1.0 - beta) * h + beta * hw, 0.0)
        part = jnp.sum(y, axis=0, keepdims=True)

        @pl.when(i == 0)
        def _():
            acc_ref[...] = part

        @pl.when(i > 0)
        def _():
            acc_ref[...] = acc_ref[...] + part

        @pl.when(i == grid - 1)
        def _():
            out_ref[...] = (
                jnp.dot(acc_ref[...] * (1.0 / _N), w1_ref[...],
                        preferred_element_type=jnp.float32) + b1_ref[...]
            )

    return pl.pallas_call(
        body,
        grid=(grid,),
        in_specs=[
            pl.BlockSpec((_NC, _BN, _H), lambda i: (0, i, 0)),
            pl.BlockSpec((_NC, _BN, _H), lambda i: (0, i, 0)),
            pl.BlockSpec((_D, _D), lambda i: (0, 0)),
            pl.BlockSpec((_D, _D), lambda i: (0, 0)),
            pl.BlockSpec((1, _D), lambda i: (0, 0)),
        ],
        out_specs=pl.BlockSpec((1, _D), lambda i: (0, 0)),
        out_shape=jax.ShapeDtypeStruct((1, _D), jnp.float32),
        scratch_shapes=[pltpu.VMEM((1, _D), jnp.float32)],
    )(agg2, x02, w, W1, b1.reshape(1, _D))


def kernel(input_x, edge_index, W0, b0, W1, b1, conv_w):
    src = edge_index[0]
    dst = edge_index[1]
    pad = _EPAD - _E
    srcp = jnp.concatenate([src, jnp.zeros((pad,), jnp.int32)])
    dstp = jnp.concatenate([dst, jnp.full((pad,), _N, jnp.int32)])
    src2 = jnp.stack([srcp, srcp + _N]).reshape(_NC, _EB, _CHUNK)
    dst2 = dstp.reshape(_EB, _CHUNK)

    x2 = _proj_tc(input_x, W0, b0)
    x02 = x2
    y = None
    for layer in range(_LAYERS):
        beta = math.log(_THETA / (layer + 1) + 1.0)
        agg2 = _segment_sum_sc(x2.reshape(_NC * _N, _H), src2, dst2)
        if layer < _LAYERS - 1:
            x2 = _combine_tc(agg2, x02, conv_w[layer], beta)
        else:
            y = _final_tc(agg2, x02, conv_w[layer], W1, b1, beta)
    return y.reshape(_D)
